# baseline (speedup 1.0000x reference)
import jax
import jax.numpy as jnp
from jax import lax
from jax.experimental import pallas as pl
from jax.experimental.pallas import tpu as pltpu
from jax.experimental.pallas import tpu_sc as plsc

BATCH = 16384
FEAT = 129
NFEAT = FEAT - 1
EMBED_DIM = 64
OUT_DIM = NFEAT + EMBED_DIM

NC = 2
NS = 16
L = 16
NW = NC * NS

ROWS_PER_W = BATCH // NW
CHUNK = 128
NCHUNK = ROWS_PER_W // CHUNK


def _sc_body(inputs_hbm, flat_hbm, table_hbm, out_hbm,
             in_v, pos_v, idxf_v, idx_v, emb_v, sem):
    wid = lax.axis_index("s") * NC + lax.axis_index("c")
    base = wid * ROWS_PER_W
    for c in range(NCHUNK):
        rb = base + c * CHUNK
        pltpu.sync_copy(inputs_hbm.at[pl.ds(rb, CHUNK), pl.ds(0, NFEAT)], in_v)
        colbase = rb * FEAT + NFEAT
        for j in range(CHUNK // L):
            pos_v[pl.ds(j * L, L)] = (
                lax.iota(jnp.int32, L) * FEAT + (j * L * FEAT) + colbase)
        pltpu.async_copy(flat_hbm.at[pos_v], idxf_v, sem).wait()
        for j in range(CHUNK // L):
            idx_v[pl.ds(j * L, L)] = idxf_v[pl.ds(j * L, L)].astype(jnp.int32)
        pltpu.async_copy(table_hbm.at[idx_v], emb_v, sem).wait()
        pltpu.sync_copy(in_v,
                        out_hbm.at[pl.ds(rb, CHUNK), pl.ds(0, NFEAT)])
        pltpu.sync_copy(emb_v,
                        out_hbm.at[pl.ds(rb, CHUNK), pl.ds(NFEAT, EMBED_DIM)])


@jax.jit
def _personalized_input(inputs, table):
    mesh = plsc.VectorSubcoreMesh(
        core_axis_name="c", subcore_axis_name="s",
        num_cores=NC, num_subcores=NS)
    return pl.kernel(
        _sc_body,
        out_type=jax.ShapeDtypeStruct((BATCH, OUT_DIM), jnp.float32),
        mesh=mesh,
        compiler_params=pltpu.CompilerParams(use_tc_tiling_on_sc=False),
        scratch_types=[
            pltpu.VMEM((CHUNK, NFEAT), jnp.float32),
            pltpu.VMEM((CHUNK,), jnp.int32),
            pltpu.VMEM((CHUNK,), jnp.float32),
            pltpu.VMEM((CHUNK,), jnp.int32),
            pltpu.VMEM((CHUNK, EMBED_DIM), jnp.float32),
            pltpu.SemaphoreType.DMA,
        ],
    )(inputs, inputs.reshape(-1), table)


def kernel(inputs, table):
    return _personalized_input(inputs, table)
